# Initial kernel scaffold; baseline (speedup 1.0000x reference)
import jax
import jax.numpy as jnp
from jax.experimental import pallas as pl


def kernel(tensor, values, indices):
    raise NotImplementedError("write your pallas kernel here")


# speedup vs baseline: 31.1417x; 31.1417x over previous
import functools

import jax
import jax.numpy as jnp
from jax import lax
from jax.experimental import pallas as pl
from jax.experimental.pallas import tpu as pltpu
from jax.experimental.pallas import tpu_sc as plsc

SHAPE = (4096, 4096)
NUMEL = SHAPE[0] * SHAPE[1]
K = 1677721

NC = 2
NS = 16
NW = NC * NS

CH = 65536
NCHUNK = NUMEL // CH
ROUNDS = NCHUNK // NW
VB = 4096
K_MAIN = K - (K % 8)
NSTART = NCHUNK + 1
NSTART_PAD = ((NSTART + 15) // 16) * 16 + 8


def _sc_body(tensor_hbm, values_hbm, indices_hbm, starts_hbm, tailix_hbm,
             tailval_hbm, out_hbm, acc, idxb, valb, stv, tiv, tvv, sem):
    def _copy(src_ref, dst_ref):
        pltpu.async_copy(src_ref, dst_ref, sem).wait()
    cidx = lax.axis_index("c")
    sidx = lax.axis_index("s")
    wid = sidx * NC + cidx
    lanes = lax.iota(jnp.int32, 16)

    _copy(starts_hbm.at[pl.ds(pl.multiple_of(wid * ROUNDS, 8), 16)],
                    stv)
    _copy(tailix_hbm, tiv)
    _copy(tailval_hbm, tvv)
    sv = stv[...]
    tail_i = tiv[...]
    tail_v = tvv[...]

    for r in range(ROUNDS):
        chunk = wid * ROUNDS + r
        lo = pl.multiple_of(chunk * CH, CH)
        p0 = sv[r]
        p1 = sv[r + 1]
        p1e = jnp.minimum(p1, K_MAIN)

        _copy(tensor_hbm.at[pl.ds(lo, CH)], acc)

        sbase0 = pl.multiple_of(
            jnp.maximum(jnp.minimum(p0 & -8, K_MAIN - VB), 0), 8)
        nb = jnp.maximum((p1e - sbase0 + (VB - 1)) // VB, 0)

        def batch_body(i, carry, _p0=p0, _p1e=p1e, _lo=lo, _sbase0=sbase0):
            ustart = _sbase0 + i * VB
            sbase = pl.multiple_of(jnp.minimum(ustart, K_MAIN - VB), 8)
            _copy(values_hbm.at[pl.ds(sbase, VB)], valb)
            _copy(indices_hbm.at[pl.ds(sbase, VB)], idxb)
            cur = jnp.maximum(_p0, ustart)

            def inner(j, carry2):
                off = pl.multiple_of(j * 16, 16)
                iv = idxb[pl.ds(off, 16)]
                vv = valb[pl.ds(off, 16)]
                pos = sbase + j * 16 + lanes
                m = (pos >= cur) & (pos < _p1e)
                liv = jnp.where(m, iv - _lo, 0)
                plsc.addupdate_scatter(acc, [liv], vv, mask=m)
                return carry2

            lax.fori_loop(0, VB // 16, inner, 0)
            return carry

        lax.fori_loop(0, nb, batch_body, 0)

        tm = (tail_i >= lo) & (tail_i < lo + CH)
        plsc.addupdate_scatter(acc, [jnp.where(tm, tail_i - lo, 0)], tail_v,
                               mask=tm)

        _copy(acc, out_hbm.at[pl.ds(lo, CH)])


_sc_call = functools.partial(
    pl.kernel,
    out_type=jax.ShapeDtypeStruct((NUMEL,), jnp.float32),
    mesh=plsc.VectorSubcoreMesh(core_axis_name="c", subcore_axis_name="s"),
    compiler_params=pltpu.CompilerParams(needs_layout_passes=False),
    scratch_types=[
        pltpu.VMEM((CH,), jnp.float32),
        pltpu.VMEM((VB,), jnp.int32),
        pltpu.VMEM((VB,), jnp.float32),
        pltpu.VMEM((16,), jnp.int32),
        pltpu.VMEM((16,), jnp.int32),
        pltpu.VMEM((16,), jnp.float32),
        pltpu.SemaphoreType.DMA,
    ],
)(_sc_body)


def kernel(tensor, values, indices):
    bounds = jnp.arange(NSTART, dtype=jnp.int32) * CH
    starts = jnp.searchsorted(indices, bounds).astype(jnp.int32)
    starts_p = jnp.concatenate(
        [starts, jnp.full((NSTART_PAD - NSTART,), K, jnp.int32)])
    ntail = K - K_MAIN
    tail_i = jnp.full((16,), -1, jnp.int32).at[:ntail].set(indices[K_MAIN:])
    tail_v = jnp.zeros((16,), jnp.float32).at[:ntail].set(values[K_MAIN:])
    out_flat = _sc_call(tensor.reshape(-1), values, indices, starts_p,
                        tail_i, tail_v)
    return out_flat.reshape(tensor.shape)


# speedup vs baseline: 41.4255x; 1.3302x over previous
import functools

import jax
import jax.numpy as jnp
from jax import lax
from jax.experimental import pallas as pl
from jax.experimental.pallas import tpu as pltpu
from jax.experimental.pallas import tpu_sc as plsc

SHAPE = (4096, 4096)
NUMEL = SHAPE[0] * SHAPE[1]
K = 1677721

NC = 2
NS = 16
NW = NC * NS

CH = 65536
ROWS_PER_CHUNK = CH // SHAPE[1]
NCHUNK = NUMEL // CH
ROUNDS = NCHUNK // NW
VB = 4096
K_MAIN = K - (K % 8)
NSTART = NCHUNK + 1
NSTART_PAD = ((NSTART + 15) // 16) * 16 + 8


def _sc_body(tensor_hbm, values_hbm, indices_hbm, starts_hbm, tailix_hbm,
             tailval_hbm, out_hbm, acc, idxb, valb, stv, tiv, tvv, sem):
    def _copy(src_ref, dst_ref):
        pltpu.async_copy(src_ref, dst_ref, sem).wait()
    cidx = lax.axis_index("c")
    sidx = lax.axis_index("s")
    wid = sidx * NC + cidx
    lanes = lax.iota(jnp.int32, 16)

    _copy(starts_hbm.at[pl.ds(pl.multiple_of(wid * ROUNDS, 8), 16)],
                    stv)
    _copy(tailix_hbm, tiv)
    _copy(tailval_hbm, tvv)
    sv = stv[...]
    tail_i = tiv[...]
    tail_v = tvv[...]

    for r in range(ROUNDS):
        chunk = wid * ROUNDS + r
        lo = pl.multiple_of(chunk * CH, CH)
        p0 = sv[r]
        p1 = sv[r + 1]
        p1e = jnp.minimum(p1, K_MAIN)

        row0 = pl.multiple_of(chunk * ROWS_PER_CHUNK, ROWS_PER_CHUNK)
        _copy(tensor_hbm.at[pl.ds(row0, ROWS_PER_CHUNK), :], acc)

        sbase0 = pl.multiple_of(
            jnp.maximum(jnp.minimum(p0 & -8, K_MAIN - VB), 0), 8)
        nb = jnp.maximum((p1e - sbase0 + (VB - 1)) // VB, 0)

        def batch_body(i, carry, _p0=p0, _p1e=p1e, _lo=lo, _sbase0=sbase0):
            ustart = _sbase0 + i * VB
            sbase = pl.multiple_of(jnp.minimum(ustart, K_MAIN - VB), 8)
            _copy(values_hbm.at[pl.ds(sbase, VB)], valb)
            _copy(indices_hbm.at[pl.ds(sbase, VB)], idxb)
            cur = jnp.maximum(_p0, ustart)

            def inner(j, carry2):
                off = pl.multiple_of(j * 16, 16)
                iv = idxb[pl.ds(off, 16)]
                vv = valb[pl.ds(off, 16)]
                pos = sbase + j * 16 + lanes
                m = (pos >= cur) & (pos < _p1e)
                liv = jnp.where(m, iv - _lo, 0)
                plsc.addupdate_scatter(
                    acc, [liv >> 12, liv & 4095], vv, mask=m)
                return carry2

            lax.fori_loop(0, VB // 16, inner, 0)
            return carry

        lax.fori_loop(0, nb, batch_body, 0)

        tm = (tail_i >= lo) & (tail_i < lo + CH)
        tl = jnp.where(tm, tail_i - lo, 0)
        plsc.addupdate_scatter(acc, [tl >> 12, tl & 4095], tail_v, mask=tm)

        _copy(acc, out_hbm.at[pl.ds(row0, ROWS_PER_CHUNK), :])


_sc_call = functools.partial(
    pl.kernel,
    out_type=jax.ShapeDtypeStruct(SHAPE, jnp.float32),
    mesh=plsc.VectorSubcoreMesh(core_axis_name="c", subcore_axis_name="s"),
    compiler_params=pltpu.CompilerParams(needs_layout_passes=False),
    scratch_types=[
        pltpu.VMEM((ROWS_PER_CHUNK, SHAPE[1]), jnp.float32),
        pltpu.VMEM((VB,), jnp.int32),
        pltpu.VMEM((VB,), jnp.float32),
        pltpu.VMEM((16,), jnp.int32),
        pltpu.VMEM((16,), jnp.int32),
        pltpu.VMEM((16,), jnp.float32),
        pltpu.SemaphoreType.DMA,
    ],
)(_sc_body)


def kernel(tensor, values, indices):
    bounds = jnp.arange(NSTART, dtype=jnp.int32) * CH
    starts = jnp.searchsorted(indices, bounds).astype(jnp.int32)
    starts_p = jnp.concatenate(
        [starts, jnp.full((NSTART_PAD - NSTART,), K, jnp.int32)])
    ntail = K - K_MAIN
    tail_i = jnp.full((16,), -1, jnp.int32).at[:ntail].set(indices[K_MAIN:])
    tail_v = jnp.zeros((16,), jnp.float32).at[:ntail].set(values[K_MAIN:])
    return _sc_call(tensor, values, indices, starts_p, tail_i, tail_v)


# speedup vs baseline: 87.2079x; 2.1052x over previous
import functools

import jax
import jax.numpy as jnp
from jax import lax
from jax.experimental import pallas as pl
from jax.experimental.pallas import tpu as pltpu
from jax.experimental.pallas import tpu_sc as plsc

SHAPE = (4096, 4096)
NUMEL = SHAPE[0] * SHAPE[1]
K = 1677721

NC = 2
NS = 16
NW = NC * NS

CH = 65536
ROWS_PER_CHUNK = CH // SHAPE[1]
NCHUNK = NUMEL // CH
ROUNDS = NCHUNK // NW
VB = 4096
K_MAIN = K - (K % 8)
BS_ITERS = K.bit_length()
COL_BITS = SHAPE[1].bit_length() - 1
COL_MASK = SHAPE[1] - 1


def _sc_body(tensor_hbm, values_hbm, indices_hbm, tailix_hbm, tailval_hbm,
             out_hbm, acc, idxb, valb, stv, midb, gatb, tiv, tvv, sem):
    def _copy(src_ref, dst_ref):
        pltpu.async_copy(src_ref, dst_ref, sem).wait()

    cidx = lax.axis_index("c")
    sidx = lax.axis_index("s")
    wid = sidx * NC + cidx
    lanes = lax.iota(jnp.int32, 16)

    _copy(tailix_hbm, tiv)
    _copy(tailval_hbm, tvv)
    tail_i = tiv[...]
    tail_v = tvv[...]

    targets = (wid * ROUNDS + lanes) * CH

    def bs_body(it, carry):
        lo_v, hi_v = carry
        active = lo_v < hi_v
        mid = lo_v + ((hi_v - lo_v) >> 1)
        midb[...] = jnp.minimum(mid, K - 1)
        _copy(indices_hbm.at[midb], gatb)
        go_right = gatb[...] < targets
        lo_n = jnp.where(active & go_right, mid + 1, lo_v)
        hi_n = jnp.where(active & (~go_right), mid, hi_v)
        return lo_n, hi_n

    lo_v, _ = lax.fori_loop(
        0, BS_ITERS, bs_body,
        (jnp.zeros((16,), jnp.int32), jnp.full((16,), K, jnp.int32)))
    stv[...] = lo_v
    sv = stv[...]

    for r in range(ROUNDS):
        chunk = wid * ROUNDS + r
        lo = pl.multiple_of(chunk * CH, CH)
        p0 = sv[r]
        p1 = sv[r + 1]
        p1e = jnp.minimum(p1, K_MAIN)

        row0 = pl.multiple_of(chunk * ROWS_PER_CHUNK, ROWS_PER_CHUNK)
        _copy(tensor_hbm.at[pl.ds(row0, ROWS_PER_CHUNK), :], acc)

        sbase0 = pl.multiple_of(
            jnp.maximum(jnp.minimum(p0 & -8, K_MAIN - VB), 0), 8)
        nb = jnp.maximum((p1e - sbase0 + (VB - 1)) // VB, 0)

        def batch_body(i, carry, _p0=p0, _p1e=p1e, _lo=lo, _sbase0=sbase0):
            ustart = _sbase0 + i * VB
            sbase = pl.multiple_of(jnp.minimum(ustart, K_MAIN - VB), 8)
            _copy(values_hbm.at[pl.ds(sbase, VB)], valb)
            _copy(indices_hbm.at[pl.ds(sbase, VB)], idxb)
            cur = jnp.maximum(_p0, ustart)

            def inner(j, carry2):
                off = pl.multiple_of(j * 16, 16)
                iv = idxb[pl.ds(off, 16)]
                vv = valb[pl.ds(off, 16)]
                pos = sbase + j * 16 + lanes
                m = (pos >= cur) & (pos < _p1e)
                liv = jnp.where(m, iv - _lo, 0)
                plsc.addupdate_scatter(
                    acc, [liv >> COL_BITS, liv & COL_MASK], vv, mask=m)
                return carry2

            lax.fori_loop(0, VB // 16, inner, 0)
            return carry

        lax.fori_loop(0, nb, batch_body, 0)

        tm = (tail_i >= lo) & (tail_i < lo + CH)
        tl = jnp.where(tm, tail_i - lo, 0)
        plsc.addupdate_scatter(acc, [tl >> COL_BITS, tl & COL_MASK], tail_v,
                               mask=tm)

        _copy(acc, out_hbm.at[pl.ds(row0, ROWS_PER_CHUNK), :])


_sc_call = functools.partial(
    pl.kernel,
    out_type=jax.ShapeDtypeStruct(SHAPE, jnp.float32),
    mesh=plsc.VectorSubcoreMesh(core_axis_name="c", subcore_axis_name="s"),
    compiler_params=pltpu.CompilerParams(needs_layout_passes=False),
    scratch_types=[
        pltpu.VMEM((ROWS_PER_CHUNK, SHAPE[1]), jnp.float32),
        pltpu.VMEM((VB,), jnp.int32),
        pltpu.VMEM((VB,), jnp.float32),
        pltpu.VMEM((16,), jnp.int32),
        pltpu.VMEM((16,), jnp.int32),
        pltpu.VMEM((16,), jnp.int32),
        pltpu.VMEM((16,), jnp.int32),
        pltpu.VMEM((16,), jnp.float32),
        pltpu.SemaphoreType.DMA,
    ],
)(_sc_body)


def kernel(tensor, values, indices):
    ntail = K - K_MAIN
    tail_i = jnp.full((16,), -1, jnp.int32).at[:ntail].set(indices[K_MAIN:])
    tail_v = jnp.zeros((16,), jnp.float32).at[:ntail].set(values[K_MAIN:])
    return _sc_call(tensor, values, indices, tail_i, tail_v)


# speedup vs baseline: 117.1724x; 1.3436x over previous
import functools

import jax
import jax.numpy as jnp
from jax import lax
from jax.experimental import pallas as pl
from jax.experimental.pallas import tpu as pltpu
from jax.experimental.pallas import tpu_sc as plsc

SHAPE = (4096, 4096)
NUMEL = SHAPE[0] * SHAPE[1]
K = 1677721

NC = 2
NS = 16
NW = NC * NS

CH = 32768
ROWS_PER_CHUNK = CH // SHAPE[1]
NCHUNK = NUMEL // CH
ROUNDS = NCHUNK // NW
NBUF = 3
VB = 4096
K_MAIN = K - (K % 8)
BS_ITERS = K.bit_length()
COL_BITS = SHAPE[1].bit_length() - 1
COL_MASK = SHAPE[1] - 1
NBOUND = ROUNDS + 1
BS_PASSES = (NBOUND + 15) // 16


def _sc_body(tensor_hbm, values_hbm, indices_hbm, tailix_hbm, tailval_hbm,
             out_hbm, acc0, acc1, acc2, idxb, valb, stv0, stv1, midb, gatb,
             midb2, gatb2, tiv, tvv, isem0, isem1, isem2, osem0, osem1,
             osem2, msem):
    accs = (acc0, acc1, acc2)
    isems = (isem0, isem1, isem2)
    osems = (osem0, osem1, osem2)
    stvs = (stv0, stv1)

    def _copy(src_ref, dst_ref):
        pltpu.async_copy(src_ref, dst_ref, msem).wait()

    cidx = lax.axis_index("c")
    sidx = lax.axis_index("s")
    wid = sidx * NC + cidx
    lanes = lax.iota(jnp.int32, 16)

    def chunk_row0(r):
        return pl.multiple_of((wid * ROUNDS + r) * ROWS_PER_CHUNK,
                              ROWS_PER_CHUNK)

    for r in range(min(2, ROUNDS)):
        pltpu.make_async_copy(
            tensor_hbm.at[pl.ds(chunk_row0(r), ROWS_PER_CHUNK), :],
            accs[r % NBUF], isems[r % NBUF]).start()

    _copy(tailix_hbm, tiv)
    _copy(tailval_hbm, tvv)
    tail_i = tiv[...]
    tail_v = tvv[...]

    targets1 = (wid * ROUNDS + lanes) * CH
    targets2 = ((wid + 1) * ROUNDS) * CH + jnp.zeros((16,), jnp.int32)

    def bs_body(it, carry):
        lo1, hi1, lo2, hi2 = carry
        a1 = lo1 < hi1
        a2 = lo2 < hi2
        mid1 = lo1 + ((hi1 - lo1) >> 1)
        mid2 = lo2 + ((hi2 - lo2) >> 1)
        midb[...] = jnp.minimum(mid1, K - 1)
        midb2[...] = jnp.minimum(mid2, K - 1)
        c1 = pltpu.async_copy(indices_hbm.at[midb], gatb, msem)
        c2 = pltpu.async_copy(indices_hbm.at[midb2], gatb2, msem)
        c1.wait()
        c2.wait()
        gr1 = gatb[...] < targets1
        gr2 = gatb2[...] < targets2
        return (jnp.where(a1 & gr1, mid1 + 1, lo1),
                jnp.where(a1 & (~gr1), mid1, hi1),
                jnp.where(a2 & gr2, mid2 + 1, lo2),
                jnp.where(a2 & (~gr2), mid2, hi2))

    z16 = jnp.zeros((16,), jnp.int32)
    k16 = jnp.full((16,), K, jnp.int32)
    lo1_v, _, lo2_v, _ = lax.fori_loop(
        0, BS_ITERS, bs_body, (z16, k16, z16, k16))
    stvs[0][...] = lo1_v
    stvs[1][...] = lo2_v

    sv0 = stvs[0][...]
    sv1 = stvs[1][...]

    def bound(i):
        return sv0[i] if i < 16 else sv1[0]

    for r in range(ROUNDS):
        b = r % NBUF
        acc = accs[b]
        chunk = wid * ROUNDS + r
        lo = pl.multiple_of(chunk * CH, CH)
        p0 = bound(r)
        p1 = bound(r + 1)
        p1e = jnp.minimum(p1, K_MAIN)

        pltpu.make_async_copy(
            tensor_hbm.at[pl.ds(chunk_row0(r), ROWS_PER_CHUNK), :],
            acc, isems[b]).wait()

        sbase0 = pl.multiple_of(
            jnp.maximum(jnp.minimum(p0 & -8, K_MAIN - VB), 0), 8)
        nb = jnp.maximum((p1e - sbase0 + (VB - 1)) // VB, 0)

        def batch_body(i, carry, _p0=p0, _p1e=p1e, _lo=lo, _sbase0=sbase0,
                       _acc=acc):
            ustart = _sbase0 + i * VB
            sbase = pl.multiple_of(jnp.minimum(ustart, K_MAIN - VB), 8)
            vcp = pltpu.async_copy(values_hbm.at[pl.ds(sbase, VB)], valb,
                                   msem)
            icp = pltpu.async_copy(indices_hbm.at[pl.ds(sbase, VB)], idxb,
                                   msem)
            vcp.wait()
            icp.wait()
            cur = jnp.maximum(_p0, ustart)

            def inner(j, carry2):
                off = pl.multiple_of(j * 16, 16)
                iv = idxb[pl.ds(off, 16)]
                vv = valb[pl.ds(off, 16)]
                pos = sbase + j * 16 + lanes
                m = (pos >= cur) & (pos < _p1e)
                liv = jnp.where(m, iv - _lo, 0)
                plsc.addupdate_scatter(
                    _acc, [liv >> COL_BITS, liv & COL_MASK], vv, mask=m)
                return carry2

            lax.fori_loop(0, VB // 16, inner, 0)
            return carry

        lax.fori_loop(0, nb, batch_body, 0)

        tm = (tail_i >= lo) & (tail_i < lo + CH)
        tl = jnp.where(tm, tail_i - lo, 0)
        plsc.addupdate_scatter(acc, [tl >> COL_BITS, tl & COL_MASK], tail_v,
                               mask=tm)

        pltpu.make_async_copy(
            acc, out_hbm.at[pl.ds(chunk_row0(r), ROWS_PER_CHUNK), :],
            osems[b]).start()

        if r + 2 < ROUNDS:
            nb2 = (r + 2) % NBUF
            if r >= 1:
                pltpu.make_async_copy(
                    accs[nb2],
                    out_hbm.at[pl.ds(chunk_row0(r - 1), ROWS_PER_CHUNK), :],
                    osems[nb2]).wait()
            pltpu.make_async_copy(
                tensor_hbm.at[pl.ds(chunk_row0(r + 2), ROWS_PER_CHUNK), :],
                accs[nb2], isems[nb2]).start()

    for r in range(max(ROUNDS - NBUF, 0), ROUNDS):
        b = r % NBUF
        pltpu.make_async_copy(
            accs[b], out_hbm.at[pl.ds(chunk_row0(r), ROWS_PER_CHUNK), :],
            osems[b]).wait()


_sc_call = functools.partial(
    pl.kernel,
    out_type=jax.ShapeDtypeStruct(SHAPE, jnp.float32),
    mesh=plsc.VectorSubcoreMesh(core_axis_name="c", subcore_axis_name="s"),
    compiler_params=pltpu.CompilerParams(needs_layout_passes=False),
    scratch_types=[
        pltpu.VMEM((ROWS_PER_CHUNK, SHAPE[1]), jnp.float32),
        pltpu.VMEM((ROWS_PER_CHUNK, SHAPE[1]), jnp.float32),
        pltpu.VMEM((ROWS_PER_CHUNK, SHAPE[1]), jnp.float32),
        pltpu.VMEM((VB,), jnp.int32),
        pltpu.VMEM((VB,), jnp.float32),
        pltpu.VMEM((16,), jnp.int32),
        pltpu.VMEM((16,), jnp.int32),
        pltpu.VMEM((16,), jnp.int32),
        pltpu.VMEM((16,), jnp.int32),
        pltpu.VMEM((16,), jnp.int32),
        pltpu.VMEM((16,), jnp.int32),
        pltpu.VMEM((16,), jnp.int32),
        pltpu.VMEM((16,), jnp.float32),
        pltpu.SemaphoreType.DMA,
        pltpu.SemaphoreType.DMA,
        pltpu.SemaphoreType.DMA,
        pltpu.SemaphoreType.DMA,
        pltpu.SemaphoreType.DMA,
        pltpu.SemaphoreType.DMA,
        pltpu.SemaphoreType.DMA,
    ],
)(_sc_body)


def kernel(tensor, values, indices):
    ntail = K - K_MAIN
    tail_i = jnp.full((16,), -1, jnp.int32).at[:ntail].set(indices[K_MAIN:])
    tail_v = jnp.zeros((16,), jnp.float32).at[:ntail].set(values[K_MAIN:])
    return _sc_call(tensor, values, indices, tail_i, tail_v)
